# Initial kernel scaffold; baseline (speedup 1.0000x reference)
import jax
import jax.numpy as jnp
from jax.experimental import pallas as pl


def kernel(x_pfc, W1, b1, W2, b2, W3, b3, We, be, Wf1, bf1, Wf2, bf2):
    raise NotImplementedError("write your pallas kernel here")


# speedup vs baseline: 4.1327x; 4.1327x over previous
import functools

import jax
import jax.numpy as jnp
from jax.experimental import pallas as pl
from jax.experimental.pallas import tpu as pltpu

K = 40
NEG = -3.0e38


def _encoder_body(x_ref, w1, b1, w2, b2, w3, b3, wet, web, be,
                  enc_out, a_out, b_out, sqh_out):
    x = x_ref[...]
    h = jax.nn.silu(jnp.dot(x, w1[...], preferred_element_type=jnp.float32) + b1[...])
    h = jax.nn.silu(jnp.dot(h, w2[...], preferred_element_type=jnp.float32) + b2[...])
    e = jnp.dot(h, w3[...], preferred_element_type=jnp.float32) + b3[...]
    enc_out[...] = e
    a_out[...] = jnp.dot(e, wet[...], preferred_element_type=jnp.float32) + be[...]
    b_out[...] = jnp.dot(e, web[...], preferred_element_type=jnp.float32)
    sqh_out[...] = 0.5 * jnp.sum(e * e, axis=1, keepdims=True)


def _knn_body(enc_r, et, sqh_t, a_r, bb, wf1, bf1, wf2, bf2,
              out_h, score_ref, acc_ref):
    R = enc_r.shape[0]
    n = et.shape[1]
    score_ref[...] = (
        jnp.dot(enc_r[...], et[...], preferred_element_type=jnp.float32)
        - sqh_t[...]
    )
    acc_ref[...] = jnp.zeros_like(acc_ref)
    ar = a_r[...]
    bmat = bb[...]
    iot = jax.lax.broadcasted_iota(jnp.int32, (R, n), 1)

    def body(_, carry):
        s = score_ref[...]
        m = jnp.max(s, axis=1, keepdims=True)
        eq = s == m
        im = jnp.max(jnp.where(eq, iot, -1), axis=1, keepdims=True)
        onehot = iot == im
        bsel = jnp.dot(onehot.astype(jnp.float32), bmat,
                       preferred_element_type=jnp.float32)
        acc_ref[...] += jax.nn.silu(ar + bsel)
        score_ref[...] = jnp.where(onehot, NEG, s)
        return carry

    jax.lax.fori_loop(0, K, body, 0, unroll=False)

    feats = acc_ref[...] * (1.0 / K)
    h = jax.nn.silu(jnp.dot(feats, wf1[...], preferred_element_type=jnp.float32)
                    + bf1[...])
    out_h[...] = jnp.dot(h, wf2[...], preferred_element_type=jnp.float32) + bf2[...]


@jax.jit
def kernel(x_pfc, W1, b1, W2, b2, W3, b3, We, be, Wf1, bf1, Wf2, bf2):
    n, d_in = x_pfc.shape
    H = W3.shape[1]
    wet = We[:H] - We[H:]
    web = We[H:]

    r_enc = 1000 if n % 1000 == 0 else n
    enc, a, b, sqh = pl.pallas_call(
        _encoder_body,
        grid=(n // r_enc,),
        in_specs=[
            pl.BlockSpec((r_enc, d_in), lambda i: (i, 0)),
            pl.BlockSpec(W1.shape, lambda i: (0, 0)),
            pl.BlockSpec((1, W1.shape[1]), lambda i: (0, 0)),
            pl.BlockSpec(W2.shape, lambda i: (0, 0)),
            pl.BlockSpec((1, W2.shape[1]), lambda i: (0, 0)),
            pl.BlockSpec(W3.shape, lambda i: (0, 0)),
            pl.BlockSpec((1, W3.shape[1]), lambda i: (0, 0)),
            pl.BlockSpec((H, H), lambda i: (0, 0)),
            pl.BlockSpec((H, H), lambda i: (0, 0)),
            pl.BlockSpec((1, H), lambda i: (0, 0)),
        ],
        out_specs=[
            pl.BlockSpec((r_enc, H), lambda i: (i, 0)),
            pl.BlockSpec((r_enc, H), lambda i: (i, 0)),
            pl.BlockSpec((r_enc, H), lambda i: (i, 0)),
            pl.BlockSpec((r_enc, 1), lambda i: (i, 0)),
        ],
        out_shape=[
            jax.ShapeDtypeStruct((n, H), jnp.float32),
            jax.ShapeDtypeStruct((n, H), jnp.float32),
            jax.ShapeDtypeStruct((n, H), jnp.float32),
            jax.ShapeDtypeStruct((n, 1), jnp.float32),
        ],
    )(x_pfc, W1, b1.reshape(1, -1), W2, b2.reshape(1, -1),
      W3, b3.reshape(1, -1), wet, web, be.reshape(1, -1))

    et = enc.T
    sqh_t = sqh.T

    r_knn = 400 if n % 400 == 0 else n
    h = pl.pallas_call(
        _knn_body,
        grid=(n // r_knn,),
        in_specs=[
            pl.BlockSpec((r_knn, H), lambda i: (i, 0)),
            pl.BlockSpec((H, n), lambda i: (0, 0)),
            pl.BlockSpec((1, n), lambda i: (0, 0)),
            pl.BlockSpec((r_knn, H), lambda i: (i, 0)),
            pl.BlockSpec((n, H), lambda i: (0, 0)),
            pl.BlockSpec(Wf1.shape, lambda i: (0, 0)),
            pl.BlockSpec((1, Wf1.shape[1]), lambda i: (0, 0)),
            pl.BlockSpec(Wf2.shape, lambda i: (0, 0)),
            pl.BlockSpec((1, Wf2.shape[1]), lambda i: (0, 0)),
        ],
        out_specs=pl.BlockSpec((r_knn, H), lambda i: (i, 0)),
        out_shape=jax.ShapeDtypeStruct((n, H), jnp.float32),
        scratch_shapes=[
            pltpu.VMEM((r_knn, n), jnp.float32),
            pltpu.VMEM((r_knn, H), jnp.float32),
        ],
    )(enc, et, sqh_t, a, b, Wf1, bf1.reshape(1, -1), Wf2, bf2.reshape(1, -1))

    return jnp.concatenate([h, x_pfc], axis=1)


# speedup vs baseline: 6.0170x; 1.4560x over previous
import functools

import jax
import jax.numpy as jnp
from jax.experimental import pallas as pl
from jax.experimental.pallas import tpu as pltpu

K = 40
NEG = -3.0e38


def _encoder_body(x_ref, w1, b1, w2, b2, w3, b3, wet, web, be,
                  enc_out, a_out, b_out, sqh_out):
    x = x_ref[...]
    h = jax.nn.silu(jnp.dot(x, w1[...], preferred_element_type=jnp.float32) + b1[...])
    h = jax.nn.silu(jnp.dot(h, w2[...], preferred_element_type=jnp.float32) + b2[...])
    e = jnp.dot(h, w3[...], preferred_element_type=jnp.float32) + b3[...]
    enc_out[...] = e
    a_out[...] = jnp.dot(e, wet[...], preferred_element_type=jnp.float32) + be[...]
    b_out[...] = jnp.dot(e, web[...], preferred_element_type=jnp.float32)
    sqh_out[...] = 0.5 * jnp.sum(e * e, axis=1, keepdims=True)


def _knn_body(enc_r, et, sqh_t, a_r, b_r, bb, wf1, bf1, wf2, bf2,
              out_h, score_ref, acc_ref):
    R = enc_r.shape[0]
    n = et.shape[1]
    blk = pl.program_id(0)
    iot = jax.lax.broadcasted_iota(jnp.int32, (R, n), 1)
    row_g = jax.lax.broadcasted_iota(jnp.int32, (R, n), 0) + blk * R
    s0 = (jnp.dot(enc_r[...], et[...], preferred_element_type=jnp.float32)
          - sqh_t[...])
    score_ref[...] = jnp.where(iot == row_g, NEG, s0)
    ar = a_r[...]
    acc_ref[...] = jax.nn.silu(ar + b_r[...])
    bmat = bb[...]

    def body(_, carry):
        s = score_ref[...]
        m = jnp.max(s, axis=1, keepdims=True)
        onehot = s == m
        bsel = jnp.dot(onehot.astype(jnp.float32), bmat,
                       preferred_element_type=jnp.float32)
        acc_ref[...] += jax.nn.silu(ar + bsel)
        score_ref[...] = jnp.where(onehot, NEG, s)
        return carry

    jax.lax.fori_loop(0, K - 1, body, 0, unroll=False)

    feats = acc_ref[...] * (1.0 / K)
    h = jax.nn.silu(jnp.dot(feats, wf1[...], preferred_element_type=jnp.float32)
                    + bf1[...])
    out_h[...] = jnp.dot(h, wf2[...], preferred_element_type=jnp.float32) + bf2[...]


@jax.jit
def kernel(x_pfc, W1, b1, W2, b2, W3, b3, We, be, Wf1, bf1, Wf2, bf2):
    n, d_in = x_pfc.shape
    H = W3.shape[1]
    wet = We[:H] - We[H:]
    web = We[H:]

    r_enc = 1000 if n % 1000 == 0 else n
    enc, a, b, sqh = pl.pallas_call(
        _encoder_body,
        grid=(n // r_enc,),
        in_specs=[
            pl.BlockSpec((r_enc, d_in), lambda i: (i, 0)),
            pl.BlockSpec(W1.shape, lambda i: (0, 0)),
            pl.BlockSpec((1, W1.shape[1]), lambda i: (0, 0)),
            pl.BlockSpec(W2.shape, lambda i: (0, 0)),
            pl.BlockSpec((1, W2.shape[1]), lambda i: (0, 0)),
            pl.BlockSpec(W3.shape, lambda i: (0, 0)),
            pl.BlockSpec((1, W3.shape[1]), lambda i: (0, 0)),
            pl.BlockSpec((H, H), lambda i: (0, 0)),
            pl.BlockSpec((H, H), lambda i: (0, 0)),
            pl.BlockSpec((1, H), lambda i: (0, 0)),
        ],
        out_specs=[
            pl.BlockSpec((r_enc, H), lambda i: (i, 0)),
            pl.BlockSpec((r_enc, H), lambda i: (i, 0)),
            pl.BlockSpec((r_enc, H), lambda i: (i, 0)),
            pl.BlockSpec((r_enc, 1), lambda i: (i, 0)),
        ],
        out_shape=[
            jax.ShapeDtypeStruct((n, H), jnp.float32),
            jax.ShapeDtypeStruct((n, H), jnp.float32),
            jax.ShapeDtypeStruct((n, H), jnp.float32),
            jax.ShapeDtypeStruct((n, 1), jnp.float32),
        ],
    )(x_pfc, W1, b1.reshape(1, -1), W2, b2.reshape(1, -1),
      W3, b3.reshape(1, -1), wet, web, be.reshape(1, -1))

    et = enc.T
    sqh_t = sqh.T

    r_knn = 400 if n % 400 == 0 else n
    h = pl.pallas_call(
        _knn_body,
        grid=(n // r_knn,),
        in_specs=[
            pl.BlockSpec((r_knn, H), lambda i: (i, 0)),
            pl.BlockSpec((H, n), lambda i: (0, 0)),
            pl.BlockSpec((1, n), lambda i: (0, 0)),
            pl.BlockSpec((r_knn, H), lambda i: (i, 0)),
            pl.BlockSpec((r_knn, H), lambda i: (i, 0)),
            pl.BlockSpec((n, H), lambda i: (0, 0)),
            pl.BlockSpec(Wf1.shape, lambda i: (0, 0)),
            pl.BlockSpec((1, Wf1.shape[1]), lambda i: (0, 0)),
            pl.BlockSpec(Wf2.shape, lambda i: (0, 0)),
            pl.BlockSpec((1, Wf2.shape[1]), lambda i: (0, 0)),
        ],
        out_specs=pl.BlockSpec((r_knn, H), lambda i: (i, 0)),
        out_shape=jax.ShapeDtypeStruct((n, H), jnp.float32),
        scratch_shapes=[
            pltpu.VMEM((r_knn, n), jnp.float32),
            pltpu.VMEM((r_knn, H), jnp.float32),
        ],
    )(enc, et, sqh_t, a, b, b, Wf1, bf1.reshape(1, -1), Wf2, bf2.reshape(1, -1))

    return jnp.concatenate([h, x_pfc], axis=1)
